# baseline (speedup 1.0000x reference)
import functools

import jax
import jax.numpy as jnp
from jax import lax
from jax.experimental import pallas as pl
from jax.experimental.pallas import tpu as pltpu
from jax.experimental.pallas import tpu_sc as plsc

N = 10000
E = 320000
IN = 128
H = 256
OUT = 64
LAYERS = 3
G = 64

NSUB = 16
NPAD = 10240
ROWS_PER_SUB = NPAD // NSUB
CH = 128
CW = 128
EPAD = 327680

BLK = 1000
NBLK = N // BLK


GC = 16


@functools.cache
def _make_seg_sum(split_edges):
    mesh = plsc.VectorSubcoreMesh(core_axis_name="c", subcore_axis_name="s",
                                  num_cores=2, num_subcores=NSUB)
    ep = EPAD // 32 if split_edges else EPAD // NSUB
    nc = ep // CH
    ng = nc // GC

    @functools.partial(
        pl.kernel,
        out_type=jax.ShapeDtypeStruct((2, NPAD, CW), jnp.float32),
        mesh=mesh,
        scratch_types=[
            pltpu.VMEM_SHARED((NPAD, CW), jnp.float32),
            pltpu.VMEM((2, GC, CH), jnp.int32),
            pltpu.VMEM((2, GC, CH), jnp.int32),
            pltpu.VMEM((CH, CW), jnp.float32),
            pltpu.VMEM((CH, CW), jnp.float32),
            pltpu.SemaphoreType.DMA,
            pltpu.SemaphoreType.DMA,
            pltpu.SemaphoreType.DMA,
            pltpu.SemaphoreType.DMA,
            pltpu.SemaphoreType.DMA,
        ],
    )
    def seg_sum(table_hbm, src_hbm, dst_hbm, out_hbm,
                acc, sidx, didx, rows0, rows1, sg0, sg1, ss0, ss1, si):
        cid = lax.axis_index("c")
        sid = lax.axis_index("s")
        if split_edges:
            srow = (cid * NSUB + sid) * nc
            drow = srow
        else:
            drow = sid * nc
            srow = cid * (EPAD // CH) + drow
        zoff = sid * ROWS_PER_SUB

        def zbody(i, _):
            r = i // (CW // 16)
            c16 = (i % (CW // 16)) * 16
            rows0[r, pl.ds(c16, 16)] = jnp.zeros((16,), jnp.float32)
            return 0
        lax.fori_loop(0, CH * (CW // 16), zbody, 0)
        for j in range(ROWS_PER_SUB // CH):
            pltpu.sync_copy(rows0, acc.at[pl.ds(zoff + j * CH, CH)])
        plsc.subcore_barrier()

        def wait_rows(sem):
            pltpu.make_async_copy(table_hbm.at[pl.ds(0, CH)], rows0, sem).wait()

        def wait_idx():
            pltpu.make_async_copy(src_hbm.at[pl.ds(srow, GC)],
                                  sidx.at[0], si).wait()

        pltpu.sync_copy(src_hbm.at[pl.ds(srow, GC)], sidx.at[0])
        pltpu.sync_copy(dst_hbm.at[pl.ds(drow, GC)], didx.at[0])
        pltpu.async_copy(table_hbm.at[sidx.at[0].at[0]], rows0, sg0)
        pltpu.async_copy(table_hbm.at[sidx.at[0].at[1]], rows1, sg1)

        for g in range(ng):
            gslot = g % 2
            sU = sidx.at[gslot]
            dU = didx.at[gslot]
            if g + 1 < ng:
                nslot = (g + 1) % 2
                pltpu.async_copy(src_hbm.at[pl.ds(srow + (g + 1) * GC, GC)],
                                 sidx.at[nslot], si)
                pltpu.async_copy(dst_hbm.at[pl.ds(drow + (g + 1) * GC, GC)],
                                 didx.at[nslot], si)

            def body(p, _, sU=sU, dU=dU):
                k0 = 2 * p
                k1 = k0 + 1

                wait_rows(sg0)
                pltpu.async_copy(rows0, acc.at[dU.at[k0]], ss0, add=True)
                wait_rows(sg1)
                pltpu.async_copy(rows1, acc.at[dU.at[k1]], ss1, add=True)

                wait_rows(ss0)

                @pl.when(k0 + 2 < GC)
                def _():
                    pltpu.async_copy(table_hbm.at[sU.at[k0 + 2]], rows0, sg0)

                wait_rows(ss1)

                @pl.when(k1 + 2 < GC)
                def _():
                    pltpu.async_copy(table_hbm.at[sU.at[k1 + 2]], rows1, sg1)
                return 0

            lax.fori_loop(0, GC // 2, body, 0)

            if g + 1 < ng:
                wait_idx()
                wait_idx()
                nxt = sidx.at[(g + 1) % 2]
                pltpu.async_copy(table_hbm.at[nxt.at[0]], rows0, sg0)
                pltpu.async_copy(table_hbm.at[nxt.at[1]], rows1, sg1)

        plsc.subcore_barrier()
        pltpu.sync_copy(acc.at[pl.ds(zoff, ROWS_PER_SUB)],
                        out_hbm.at[cid, pl.ds(zoff, ROWS_PER_SUB)])

    return seg_sum


def _k_mm1(h0_ref, h1_ref, a0_ref, a1_ref, w_ref, b_ref, eps_ref,
           y_ref, s_ref, q_ref, *, agg_sum):
    i = pl.program_id(0)
    h = jnp.concatenate([h0_ref[0], h1_ref[0]], axis=1)
    if agg_sum:
        a = a0_ref[0] + a1_ref[0]
    else:
        a = jnp.concatenate([a0_ref[0], a1_ref[0]], axis=1)
    t = h * eps_ref[0, 0] + a
    y = jnp.dot(t, w_ref[...], preferred_element_type=jnp.float32) + b_ref[...]
    y_ref[...] = y

    @pl.when(i == 0)
    def _():
        s_ref[...] = jnp.zeros_like(s_ref)
        q_ref[...] = jnp.zeros_like(q_ref)

    s_ref[...] += jnp.sum(y, axis=0, keepdims=True)
    q_ref[...] += jnp.sum(y * y, axis=0, keepdims=True)


def _mm1(h2, agg, W1, b1, epsp, c, agg_sum):
    cw = c // 2
    return pl.pallas_call(
        functools.partial(_k_mm1, agg_sum=agg_sum),
        grid=(NBLK,),
        in_specs=[
            pl.BlockSpec((1, BLK, cw), lambda i: (0, i, 0)),
            pl.BlockSpec((1, BLK, cw), lambda i: (1, i, 0)),
            pl.BlockSpec((1, BLK, CW), lambda i: (0, i, 0)),
            pl.BlockSpec((1, BLK, CW), lambda i: (1, i, 0)),
            pl.BlockSpec((c, 2 * H), lambda i: (0, 0)),
            pl.BlockSpec((1, 2 * H), lambda i: (0, 0)),
            pl.BlockSpec(memory_space=pltpu.SMEM),
        ],
        out_specs=[
            pl.BlockSpec((BLK, 2 * H), lambda i: (i, 0)),
            pl.BlockSpec((1, 2 * H), lambda i: (0, 0)),
            pl.BlockSpec((1, 2 * H), lambda i: (0, 0)),
        ],
        out_shape=[
            jax.ShapeDtypeStruct((N, 2 * H), jnp.float32),
            jax.ShapeDtypeStruct((1, 2 * H), jnp.float32),
            jax.ShapeDtypeStruct((1, 2 * H), jnp.float32),
        ],
    )(h2, h2, agg, agg, W1, b1, epsp)


def _k_mm2(y_ref, sc_ref, sh_ref, w_ref, b_ref, y2_ref, s_ref, q_ref):
    i = pl.program_id(0)
    yn = jnp.maximum(y_ref[...] * sc_ref[...] + sh_ref[...], 0.0)
    y2 = jnp.dot(yn, w_ref[...], preferred_element_type=jnp.float32) + b_ref[...]
    y2_ref[...] = y2

    @pl.when(i == 0)
    def _():
        s_ref[...] = jnp.zeros_like(s_ref)
        q_ref[...] = jnp.zeros_like(q_ref)

    s_ref[...] += jnp.sum(y2, axis=0, keepdims=True)
    q_ref[...] += jnp.sum(y2 * y2, axis=0, keepdims=True)


def _mm2(y1, sc1, sh1, W2, b2):
    return pl.pallas_call(
        _k_mm2,
        grid=(NBLK,),
        in_specs=[
            pl.BlockSpec((BLK, 2 * H), lambda i: (i, 0)),
            pl.BlockSpec((1, 2 * H), lambda i: (0, 0)),
            pl.BlockSpec((1, 2 * H), lambda i: (0, 0)),
            pl.BlockSpec((2 * H, H), lambda i: (0, 0)),
            pl.BlockSpec((1, H), lambda i: (0, 0)),
        ],
        out_specs=[
            pl.BlockSpec((BLK, H), lambda i: (i, 0)),
            pl.BlockSpec((1, H), lambda i: (0, 0)),
            pl.BlockSpec((1, H), lambda i: (0, 0)),
        ],
        out_shape=[
            jax.ShapeDtypeStruct((N, H), jnp.float32),
            jax.ShapeDtypeStruct((1, H), jnp.float32),
            jax.ShapeDtypeStruct((1, H), jnp.float32),
        ],
    )(y1, sc1, sh1, W2, b2)


def _k_norm(y2_ref, sc_ref, sh_ref, o_ref):
    o_ref[0] = jnp.maximum(y2_ref[...] * sc_ref[...] + sh_ref[...], 0.0)


def _norm(y2, sc2, sh2):
    cw = H // 2
    return pl.pallas_call(
        _k_norm,
        grid=(NBLK, 2),
        in_specs=[
            pl.BlockSpec((BLK, cw), lambda i, k: (i, k)),
            pl.BlockSpec((1, cw), lambda i, k: (0, k)),
            pl.BlockSpec((1, cw), lambda i, k: (0, k)),
        ],
        out_specs=pl.BlockSpec((1, BLK, cw), lambda i, k: (k, i, 0)),
        out_shape=jax.ShapeDtypeStruct((2, N, cw), jnp.float32),
    )(y2, sc2, sh2)


def _k_head(h0_ref, h1_ref, bt_ref, w1_ref, b1_ref, g_ref, bb_ref,
            w2_ref, b2_ref, o_ref, acc_ref):
    i = pl.program_id(0)

    @pl.when(i == 0)
    def _():
        acc_ref[...] = jnp.zeros_like(acc_ref)

    h = jnp.concatenate([h0_ref[0], h1_ref[0]], axis=1)
    bvals = bt_ref[0, 0, :]
    oh = (bvals[:, None] ==
          lax.broadcasted_iota(jnp.int32, (1, G), 1)).astype(jnp.float32)
    acc_ref[...] += lax.dot_general(
        oh, h, (((0,), (0,)), ((), ())), preferred_element_type=jnp.float32)

    @pl.when(i == pl.num_programs(0) - 1)
    def _():
        p = acc_ref[...]
        y = jnp.dot(p, w1_ref[...],
                    preferred_element_type=jnp.float32) + b1_ref[...]
        m = jnp.mean(y, axis=0, keepdims=True)
        v = jnp.mean(y * y, axis=0, keepdims=True) - m * m
        yn = jnp.maximum(
            g_ref[...] * (y - m) * lax.rsqrt(v + 1e-5) + bb_ref[...], 0.0)
        z = jnp.dot(yn, w2_ref[...],
                    preferred_element_type=jnp.float32) + b2_ref[...]
        ze = z - jnp.max(z, axis=1, keepdims=True)
        o_ref[...] = ze - jnp.log(jnp.sum(jnp.exp(ze), axis=1, keepdims=True))


def _head(h2, batch3, w1, b1, g, bb, w2, b2):
    cw = H // 2
    return pl.pallas_call(
        _k_head,
        grid=(NBLK,),
        in_specs=[
            pl.BlockSpec((1, BLK, cw), lambda i: (0, i, 0)),
            pl.BlockSpec((1, BLK, cw), lambda i: (1, i, 0)),
            pl.BlockSpec((1, 1, BLK), lambda i: (i, 0, 0)),
            pl.BlockSpec((H, H), lambda i: (0, 0)),
            pl.BlockSpec((1, H), lambda i: (0, 0)),
            pl.BlockSpec((1, H), lambda i: (0, 0)),
            pl.BlockSpec((1, H), lambda i: (0, 0)),
            pl.BlockSpec((H, OUT), lambda i: (0, 0)),
            pl.BlockSpec((1, OUT), lambda i: (0, 0)),
        ],
        out_specs=pl.BlockSpec((G, OUT), lambda i: (0, 0)),
        out_shape=jax.ShapeDtypeStruct((G, OUT), jnp.float32),
        scratch_shapes=[pltpu.VMEM((G, H), jnp.float32)],
    )(h2, h2, batch3, w1, b1, g, bb, w2, b2)


def kernel(x, edge_index, batch, params):
    src = edge_index[0]
    dst = edge_index[1]
    npad_e = EPAD - E
    pad_i = jnp.arange(npad_e, dtype=jnp.int32)
    srcp = jnp.concatenate([src, pad_i % N])
    dstp = jnp.concatenate([dst, N + pad_i % (NPAD - N)])
    srcp2d = srcp.reshape(-1, CH)
    src2p2d = jnp.concatenate([srcp, srcp + N]).reshape(-1, CH)
    dstp2d = dstp.reshape(-1, CH)
    batch3 = batch.reshape(NBLK, 1, BLK)

    cw = IN // 2
    h2 = jnp.stack([x[:, :cw], x[:, cw:]])
    for i in range(LAYERS):
        p = params[f"conv{i}"]
        if i == 0:
            agg = _make_seg_sum(True)(x, srcp2d, dstp2d)
        else:
            table = h2.reshape(2 * N, cw)
            agg = _make_seg_sum(False)(table, src2p2d, dstp2d)
        epsp = (1.0 + p["eps"]).reshape(1, 1)
        y1, s1, q1 = _mm1(h2, agg, p["W1"], p["b1"].reshape(1, -1),
                          epsp, 2 * cw, agg_sum=(i == 0))
        mu = s1 / N
        isg = lax.rsqrt(q1 / N - mu * mu + 1e-5)
        sc1 = p["g1"].reshape(1, -1) * isg
        sh1 = p["bt1"].reshape(1, -1) - mu * sc1
        y2, s2, q2 = _mm2(y1, sc1, sh1, p["W2"], p["b2"].reshape(1, -1))
        mu2 = s2 / N
        isg2 = lax.rsqrt(q2 / N - mu2 * mu2 + 1e-5)
        sc2 = params[f"bn{i}_g"].reshape(1, -1) * isg2
        sh2 = params[f"bn{i}_b"].reshape(1, -1) - mu2 * sc2
        h2 = _norm(y2, sc2, sh2)
        cw = H // 2

    return _head(h2, batch3, params["lin1_W"],
                 params["lin1_b"].reshape(1, -1),
                 params["bn1_g"].reshape(1, -1),
                 params["bn1_b"].reshape(1, -1),
                 params["lin2_W"], params["lin2_b"].reshape(1, -1))


# speedup vs baseline: 1.1872x; 1.1872x over previous
import functools

import jax
import jax.numpy as jnp
from jax import lax
from jax.experimental import pallas as pl
from jax.experimental.pallas import tpu as pltpu
from jax.experimental.pallas import tpu_sc as plsc

N = 10000
E = 320000
IN = 128
H = 256
OUT = 64
LAYERS = 3
G = 64

NSUB = 16
NPAD = 10240
ROWS_PER_SUB = NPAD // NSUB
CH = 128
CW = 128
EPAD = 327680

BLK = 1000
NBLK = N // BLK


GC = 16


@functools.cache
def _make_seg_sum(split_edges):
    mesh = plsc.VectorSubcoreMesh(core_axis_name="c", subcore_axis_name="s",
                                  num_cores=2, num_subcores=NSUB)
    ep = EPAD // 32 if split_edges else EPAD // NSUB
    nc = ep // CH
    ng = nc // GC

    @functools.partial(
        pl.kernel,
        out_type=jax.ShapeDtypeStruct((2, NPAD, CW), jnp.float32),
        mesh=mesh,
        scratch_types=[
            pltpu.VMEM_SHARED((NPAD, CW), jnp.float32),
            pltpu.VMEM((2, GC, CH), jnp.int32),
            pltpu.VMEM((2, GC, CH), jnp.int32),
            pltpu.VMEM((CH, CW), jnp.float32),
            pltpu.VMEM((CH, CW), jnp.float32),
            pltpu.SemaphoreType.DMA,
            pltpu.SemaphoreType.DMA,
            pltpu.SemaphoreType.DMA,
            pltpu.SemaphoreType.DMA,
            pltpu.SemaphoreType.DMA,
        ],
    )
    def seg_sum(table_hbm, src_hbm, dst_hbm, out_hbm,
                acc, sidx, didx, rows0, rows1, sg0, sg1, ss0, ss1, si):
        cid = lax.axis_index("c")
        sid = lax.axis_index("s")
        if split_edges:
            srow = (cid * NSUB + sid) * nc
            drow = srow
        else:
            drow = sid * nc
            srow = cid * (EPAD // CH) + drow
        zoff = sid * ROWS_PER_SUB

        def zbody(i, _):
            r = i // (CW // 16)
            c16 = (i % (CW // 16)) * 16
            rows0[r, pl.ds(c16, 16)] = jnp.zeros((16,), jnp.float32)
            return 0
        lax.fori_loop(0, CH * (CW // 16), zbody, 0)
        for j in range(ROWS_PER_SUB // CH):
            pltpu.sync_copy(rows0, acc.at[pl.ds(zoff + j * CH, CH)])
        plsc.subcore_barrier()

        def wait_rows(sem):
            pltpu.make_async_copy(table_hbm.at[pl.ds(0, CH)], rows0, sem).wait()

        def wait_idx():
            pltpu.make_async_copy(src_hbm.at[pl.ds(srow, GC)],
                                  sidx.at[0], si).wait()

        pltpu.sync_copy(src_hbm.at[pl.ds(srow, GC)], sidx.at[0])
        pltpu.sync_copy(dst_hbm.at[pl.ds(drow, GC)], didx.at[0])
        pltpu.async_copy(table_hbm.at[sidx.at[0].at[0]], rows0, sg0)
        pltpu.async_copy(table_hbm.at[sidx.at[0].at[1]], rows1, sg1)

        for g in range(ng):
            gslot = g % 2
            sU = sidx.at[gslot]
            dU = didx.at[gslot]
            if g + 1 < ng:
                nslot = (g + 1) % 2
                pltpu.async_copy(src_hbm.at[pl.ds(srow + (g + 1) * GC, GC)],
                                 sidx.at[nslot], si)
                pltpu.async_copy(dst_hbm.at[pl.ds(drow + (g + 1) * GC, GC)],
                                 didx.at[nslot], si)

            def body(p, _, sU=sU, dU=dU):
                k0 = 2 * p
                k1 = k0 + 1

                wait_rows(sg0)
                pltpu.async_copy(rows0, acc.at[dU.at[k0]], ss0, add=True)
                wait_rows(ss0)

                @pl.when(k0 + 2 < GC)
                def _():
                    pltpu.async_copy(table_hbm.at[sU.at[k0 + 2]], rows0, sg0)

                wait_rows(sg1)
                pltpu.async_copy(rows1, acc.at[dU.at[k1]], ss1, add=True)
                wait_rows(ss1)

                @pl.when(k1 + 2 < GC)
                def _():
                    pltpu.async_copy(table_hbm.at[sU.at[k1 + 2]], rows1, sg1)
                return 0

            lax.fori_loop(0, GC // 2, body, 0)

            if g + 1 < ng:
                wait_idx()
                wait_idx()
                nxt = sidx.at[(g + 1) % 2]
                pltpu.async_copy(table_hbm.at[nxt.at[0]], rows0, sg0)
                pltpu.async_copy(table_hbm.at[nxt.at[1]], rows1, sg1)

        plsc.subcore_barrier()
        pltpu.sync_copy(acc.at[pl.ds(zoff, ROWS_PER_SUB)],
                        out_hbm.at[cid, pl.ds(zoff, ROWS_PER_SUB)])

    return seg_sum


def _k_mm1pre(h0_ref, h1_ref, w_ref, b_ref, eps_ref, a_ref):
    h = jnp.concatenate([h0_ref[0], h1_ref[0]], axis=1) * eps_ref[0, 0]
    a_ref[...] = (jnp.dot(h, w_ref[...], preferred_element_type=jnp.float32)
                  + b_ref[...])


def _mm1pre(h2, W1, b1, epsp, c):
    cw = c // 2
    return pl.pallas_call(
        _k_mm1pre,
        grid=(NBLK,),
        in_specs=[
            pl.BlockSpec((1, BLK, cw), lambda i: (0, i, 0)),
            pl.BlockSpec((1, BLK, cw), lambda i: (1, i, 0)),
            pl.BlockSpec((c, 2 * H), lambda i: (0, 0)),
            pl.BlockSpec((1, 2 * H), lambda i: (0, 0)),
            pl.BlockSpec(memory_space=pltpu.SMEM),
        ],
        out_specs=pl.BlockSpec((BLK, 2 * H), lambda i: (i, 0)),
        out_shape=jax.ShapeDtypeStruct((N, 2 * H), jnp.float32),
    )(h2, h2, W1, b1, epsp)


def _k_mm1post(a_ref, g0_ref, g1_ref, w0_ref, w1_ref,
               y_ref, s_ref, q_ref, *, agg_sum):
    i = pl.program_id(0)
    if agg_sum:
        m = jnp.dot(g0_ref[0] + g1_ref[0], w0_ref[...],
                    preferred_element_type=jnp.float32)
    else:
        m = (jnp.dot(g0_ref[0], w0_ref[...],
                     preferred_element_type=jnp.float32)
             + jnp.dot(g1_ref[0], w1_ref[...],
                       preferred_element_type=jnp.float32))
    y = a_ref[...] + m
    y_ref[...] = y

    @pl.when(i == 0)
    def _():
        s_ref[...] = jnp.zeros_like(s_ref)
        q_ref[...] = jnp.zeros_like(q_ref)

    s_ref[...] += jnp.sum(y, axis=0, keepdims=True)
    q_ref[...] += jnp.sum(y * y, axis=0, keepdims=True)


def _mm1post(a, agg, W1, agg_sum):
    nwb = W1.shape[0] // CW
    return pl.pallas_call(
        functools.partial(_k_mm1post, agg_sum=agg_sum),
        grid=(NBLK,),
        in_specs=[
            pl.BlockSpec((BLK, 2 * H), lambda i: (i, 0)),
            pl.BlockSpec((1, BLK, CW), lambda i: (0, i, 0)),
            pl.BlockSpec((1, BLK, CW), lambda i: (1, i, 0)),
            pl.BlockSpec((CW, 2 * H), lambda i: (0, 0)),
            pl.BlockSpec((CW, 2 * H), lambda i: (min(nwb - 1, 1), 0)),
        ],
        out_specs=[
            pl.BlockSpec((BLK, 2 * H), lambda i: (i, 0)),
            pl.BlockSpec((1, 2 * H), lambda i: (0, 0)),
            pl.BlockSpec((1, 2 * H), lambda i: (0, 0)),
        ],
        out_shape=[
            jax.ShapeDtypeStruct((N, 2 * H), jnp.float32),
            jax.ShapeDtypeStruct((1, 2 * H), jnp.float32),
            jax.ShapeDtypeStruct((1, 2 * H), jnp.float32),
        ],
    )(a, agg, agg, W1, W1)


def _k_mm2(y_ref, sc_ref, sh_ref, w_ref, b_ref, y2_ref, s_ref, q_ref):
    i = pl.program_id(0)
    yn = jnp.maximum(y_ref[...] * sc_ref[...] + sh_ref[...], 0.0)
    y2 = jnp.dot(yn, w_ref[...], preferred_element_type=jnp.float32) + b_ref[...]
    y2_ref[...] = y2

    @pl.when(i == 0)
    def _():
        s_ref[...] = jnp.zeros_like(s_ref)
        q_ref[...] = jnp.zeros_like(q_ref)

    s_ref[...] += jnp.sum(y2, axis=0, keepdims=True)
    q_ref[...] += jnp.sum(y2 * y2, axis=0, keepdims=True)


def _mm2(y1, sc1, sh1, W2, b2):
    return pl.pallas_call(
        _k_mm2,
        grid=(NBLK,),
        in_specs=[
            pl.BlockSpec((BLK, 2 * H), lambda i: (i, 0)),
            pl.BlockSpec((1, 2 * H), lambda i: (0, 0)),
            pl.BlockSpec((1, 2 * H), lambda i: (0, 0)),
            pl.BlockSpec((2 * H, H), lambda i: (0, 0)),
            pl.BlockSpec((1, H), lambda i: (0, 0)),
        ],
        out_specs=[
            pl.BlockSpec((BLK, H), lambda i: (i, 0)),
            pl.BlockSpec((1, H), lambda i: (0, 0)),
            pl.BlockSpec((1, H), lambda i: (0, 0)),
        ],
        out_shape=[
            jax.ShapeDtypeStruct((N, H), jnp.float32),
            jax.ShapeDtypeStruct((1, H), jnp.float32),
            jax.ShapeDtypeStruct((1, H), jnp.float32),
        ],
    )(y1, sc1, sh1, W2, b2)


def _k_norm(y2_ref, sc_ref, sh_ref, o_ref):
    o_ref[0] = jnp.maximum(y2_ref[...] * sc_ref[...] + sh_ref[...], 0.0)


def _norm(y2, sc2, sh2):
    cw = H // 2
    return pl.pallas_call(
        _k_norm,
        grid=(NBLK, 2),
        in_specs=[
            pl.BlockSpec((BLK, cw), lambda i, k: (i, k)),
            pl.BlockSpec((1, cw), lambda i, k: (0, k)),
            pl.BlockSpec((1, cw), lambda i, k: (0, k)),
        ],
        out_specs=pl.BlockSpec((1, BLK, cw), lambda i, k: (k, i, 0)),
        out_shape=jax.ShapeDtypeStruct((2, N, cw), jnp.float32),
    )(y2, sc2, sh2)


def _k_head(h0_ref, h1_ref, bt_ref, w1_ref, b1_ref, g_ref, bb_ref,
            w2_ref, b2_ref, o_ref, acc_ref):
    i = pl.program_id(0)

    @pl.when(i == 0)
    def _():
        acc_ref[...] = jnp.zeros_like(acc_ref)

    h = jnp.concatenate([h0_ref[0], h1_ref[0]], axis=1)
    bvals = bt_ref[0, 0, :]
    oh = (bvals[:, None] ==
          lax.broadcasted_iota(jnp.int32, (1, G), 1)).astype(jnp.float32)
    acc_ref[...] += lax.dot_general(
        oh, h, (((0,), (0,)), ((), ())), preferred_element_type=jnp.float32)

    @pl.when(i == pl.num_programs(0) - 1)
    def _():
        p = acc_ref[...]
        y = jnp.dot(p, w1_ref[...],
                    preferred_element_type=jnp.float32) + b1_ref[...]
        m = jnp.mean(y, axis=0, keepdims=True)
        v = jnp.mean(y * y, axis=0, keepdims=True) - m * m
        yn = jnp.maximum(
            g_ref[...] * (y - m) * lax.rsqrt(v + 1e-5) + bb_ref[...], 0.0)
        z = jnp.dot(yn, w2_ref[...],
                    preferred_element_type=jnp.float32) + b2_ref[...]
        ze = z - jnp.max(z, axis=1, keepdims=True)
        o_ref[...] = ze - jnp.log(jnp.sum(jnp.exp(ze), axis=1, keepdims=True))


def _head(h2, batch3, w1, b1, g, bb, w2, b2):
    cw = H // 2
    return pl.pallas_call(
        _k_head,
        grid=(NBLK,),
        in_specs=[
            pl.BlockSpec((1, BLK, cw), lambda i: (0, i, 0)),
            pl.BlockSpec((1, BLK, cw), lambda i: (1, i, 0)),
            pl.BlockSpec((1, 1, BLK), lambda i: (i, 0, 0)),
            pl.BlockSpec((H, H), lambda i: (0, 0)),
            pl.BlockSpec((1, H), lambda i: (0, 0)),
            pl.BlockSpec((1, H), lambda i: (0, 0)),
            pl.BlockSpec((1, H), lambda i: (0, 0)),
            pl.BlockSpec((H, OUT), lambda i: (0, 0)),
            pl.BlockSpec((1, OUT), lambda i: (0, 0)),
        ],
        out_specs=pl.BlockSpec((G, OUT), lambda i: (0, 0)),
        out_shape=jax.ShapeDtypeStruct((G, OUT), jnp.float32),
        scratch_shapes=[pltpu.VMEM((G, H), jnp.float32)],
    )(h2, h2, batch3, w1, b1, g, bb, w2, b2)


def kernel(x, edge_index, batch, params):
    src = edge_index[0]
    dst = edge_index[1]
    npad_e = EPAD - E
    pad_i = jnp.arange(npad_e, dtype=jnp.int32)
    srcp = jnp.concatenate([src, pad_i % N])
    dstp = jnp.concatenate([dst, N + pad_i % (NPAD - N)])
    srcp2d = srcp.reshape(-1, CH)
    src2p2d = jnp.concatenate([srcp, srcp + N]).reshape(-1, CH)
    dstp2d = dstp.reshape(-1, CH)
    batch3 = batch.reshape(NBLK, 1, BLK)

    cw = IN // 2
    h2 = jnp.stack([x[:, :cw], x[:, cw:]])
    for i in range(LAYERS):
        p = params[f"conv{i}"]
        epsp = (1.0 + p["eps"]).reshape(1, 1)
        a = _mm1pre(h2, p["W1"], p["b1"].reshape(1, -1), epsp, 2 * cw)
        if i == 0:
            agg = _make_seg_sum(True)(x, srcp2d, dstp2d)
        else:
            table = h2.reshape(2 * N, cw)
            agg = _make_seg_sum(False)(table, src2p2d, dstp2d)
        y1, s1, q1 = _mm1post(a, agg, p["W1"], agg_sum=(i == 0))
        mu = s1 / N
        isg = lax.rsqrt(q1 / N - mu * mu + 1e-5)
        sc1 = p["g1"].reshape(1, -1) * isg
        sh1 = p["bt1"].reshape(1, -1) - mu * sc1
        y2, s2, q2 = _mm2(y1, sc1, sh1, p["W2"], p["b2"].reshape(1, -1))
        mu2 = s2 / N
        isg2 = lax.rsqrt(q2 / N - mu2 * mu2 + 1e-5)
        sc2 = params[f"bn{i}_g"].reshape(1, -1) * isg2
        sh2 = params[f"bn{i}_b"].reshape(1, -1) - mu2 * sc2
        h2 = _norm(y2, sc2, sh2)
        cw = H // 2

    return _head(h2, batch3, params["lin1_W"],
                 params["lin1_b"].reshape(1, -1),
                 params["bn1_g"].reshape(1, -1),
                 params["bn1_b"].reshape(1, -1),
                 params["lin2_W"], params["lin2_b"].reshape(1, -1))


# speedup vs baseline: 1.2289x; 1.0351x over previous
import functools

import jax
import jax.numpy as jnp
from jax import lax
from jax.experimental import pallas as pl
from jax.experimental.pallas import tpu as pltpu
from jax.experimental.pallas import tpu_sc as plsc

N = 10000
E = 320000
IN = 128
H = 256
OUT = 64
LAYERS = 3
G = 64

NSUB = 16
NPAD = 10240
ROWS_PER_SUB = NPAD // NSUB
CH = 128
CW = 128
EPAD = 327680

BLK = 1000
NBLK = N // BLK


@functools.cache
def _make_seg_sum(split_edges):
    mesh = plsc.VectorSubcoreMesh(core_axis_name="c", subcore_axis_name="s",
                                  num_cores=2, num_subcores=NSUB)
    ep = EPAD // 32 if split_edges else EPAD // NSUB
    nc = ep // CH
    GC = 16
    ng = nc // GC

    @functools.partial(
        pl.kernel,
        out_type=jax.ShapeDtypeStruct((2, NPAD, CW), jnp.float32),
        mesh=mesh,
        scratch_types=[
            pltpu.VMEM_SHARED((NPAD, CW), jnp.float32),
            pltpu.VMEM((2, GC, CH), jnp.int32),
            pltpu.VMEM((2, GC, CH), jnp.int32),
            pltpu.VMEM((CH, CW), jnp.float32),
            pltpu.VMEM((CH, CW), jnp.float32),
            pltpu.SemaphoreType.DMA,
            pltpu.SemaphoreType.DMA,
            pltpu.SemaphoreType.DMA,
            pltpu.SemaphoreType.DMA,
            pltpu.SemaphoreType.DMA,
        ],
    )
    def seg_sum(table_hbm, src_hbm, dst_hbm, out_hbm,
                acc, sidx, didx, rows0, rows1, sg0, sg1, ss0, ss1, si):
        cid = lax.axis_index("c")
        sid = lax.axis_index("s")
        if split_edges:
            srow = (cid * NSUB + sid) * nc
            drow = srow
        else:
            drow = sid * nc
            srow = cid * (EPAD // CH) + drow
        zoff = sid * ROWS_PER_SUB

        def wait_rows(sem):
            pltpu.make_async_copy(table_hbm.at[pl.ds(0, CH)], rows0, sem).wait()

        def wait_idx():
            pltpu.make_async_copy(src_hbm.at[pl.ds(srow, GC)],
                                  sidx.at[0], si).wait()

        pltpu.sync_copy(src_hbm.at[pl.ds(srow, GC)], sidx.at[0])
        pltpu.sync_copy(dst_hbm.at[pl.ds(drow, GC)], didx.at[0])
        pltpu.async_copy(table_hbm.at[sidx.at[0].at[0]], rows0, sg0)

        def zbody(i, _):
            r = i // (CW // 16)
            c16 = (i % (CW // 16)) * 16
            rows1[r, pl.ds(c16, 16)] = jnp.zeros((16,), jnp.float32)
            return 0
        lax.fori_loop(0, CH * (CW // 16), zbody, 0)
        for j in range(ROWS_PER_SUB // CH):
            pltpu.async_copy(rows1, acc.at[pl.ds(zoff + j * CH, CH)], si)
        for j in range(ROWS_PER_SUB // CH):
            pltpu.make_async_copy(table_hbm.at[pl.ds(0, CH)], rows1, si).wait()
        plsc.subcore_barrier()
        pltpu.async_copy(table_hbm.at[sidx.at[0].at[1]], rows1, sg1)

        for g in range(ng):
            gslot = g % 2
            sU = sidx.at[gslot]
            dU = didx.at[gslot]
            if g + 1 < ng:
                nslot = (g + 1) % 2
                pltpu.async_copy(src_hbm.at[pl.ds(srow + (g + 1) * GC, GC)],
                                 sidx.at[nslot], si)
                pltpu.async_copy(dst_hbm.at[pl.ds(drow + (g + 1) * GC, GC)],
                                 didx.at[nslot], si)

            def body(p, _, sU=sU, dU=dU):
                k0 = 2 * p
                k1 = k0 + 1

                wait_rows(sg0)
                pltpu.async_copy(rows0, acc.at[dU.at[k0]], ss0, add=True)
                wait_rows(ss0)

                @pl.when(k0 + 2 < GC)
                def _():
                    pltpu.async_copy(table_hbm.at[sU.at[k0 + 2]], rows0, sg0)

                wait_rows(sg1)
                pltpu.async_copy(rows1, acc.at[dU.at[k1]], ss1, add=True)
                wait_rows(ss1)

                @pl.when(k1 + 2 < GC)
                def _():
                    pltpu.async_copy(table_hbm.at[sU.at[k1 + 2]], rows1, sg1)
                return 0

            lax.fori_loop(0, GC // 2, body, 0)

            if g + 1 < ng:
                wait_idx()
                wait_idx()
                nxt = sidx.at[(g + 1) % 2]
                pltpu.async_copy(table_hbm.at[nxt.at[0]], rows0, sg0)
                pltpu.async_copy(table_hbm.at[nxt.at[1]], rows1, sg1)

        plsc.subcore_barrier()
        pltpu.sync_copy(acc.at[pl.ds(zoff, ROWS_PER_SUB)],
                        out_hbm.at[cid, pl.ds(zoff, ROWS_PER_SUB)])

    return seg_sum


def _k_mm1(h0_ref, h1_ref, a0_ref, a1_ref, w_ref, b_ref, eps_ref,
           y_ref, s_ref, q_ref, *, agg_sum):
    i = pl.program_id(0)
    h = jnp.concatenate([h0_ref[0], h1_ref[0]], axis=1)
    if agg_sum:
        a = a0_ref[0] + a1_ref[0]
    else:
        a = jnp.concatenate([a0_ref[0], a1_ref[0]], axis=1)
    t = h * eps_ref[0, 0] + a
    y = jnp.dot(t, w_ref[...], preferred_element_type=jnp.float32) + b_ref[...]
    y_ref[...] = y

    @pl.when(i == 0)
    def _():
        s_ref[...] = jnp.zeros_like(s_ref)
        q_ref[...] = jnp.zeros_like(q_ref)

    s_ref[...] += jnp.sum(y, axis=0, keepdims=True)
    q_ref[...] += jnp.sum(y * y, axis=0, keepdims=True)


def _mm1(h2, agg, W1, b1, epsp, c, agg_sum):
    cw = c // 2
    return pl.pallas_call(
        functools.partial(_k_mm1, agg_sum=agg_sum),
        grid=(NBLK,),
        in_specs=[
            pl.BlockSpec((1, BLK, cw), lambda i: (0, i, 0)),
            pl.BlockSpec((1, BLK, cw), lambda i: (1, i, 0)),
            pl.BlockSpec((1, BLK, CW), lambda i: (0, i, 0)),
            pl.BlockSpec((1, BLK, CW), lambda i: (1, i, 0)),
            pl.BlockSpec((c, 2 * H), lambda i: (0, 0)),
            pl.BlockSpec((1, 2 * H), lambda i: (0, 0)),
            pl.BlockSpec(memory_space=pltpu.SMEM),
        ],
        out_specs=[
            pl.BlockSpec((BLK, 2 * H), lambda i: (i, 0)),
            pl.BlockSpec((1, 2 * H), lambda i: (0, 0)),
            pl.BlockSpec((1, 2 * H), lambda i: (0, 0)),
        ],
        out_shape=[
            jax.ShapeDtypeStruct((N, 2 * H), jnp.float32),
            jax.ShapeDtypeStruct((1, 2 * H), jnp.float32),
            jax.ShapeDtypeStruct((1, 2 * H), jnp.float32),
        ],
    )(h2, h2, agg, agg, W1, b1, epsp)


def _k_mm2(y_ref, sc_ref, sh_ref, w_ref, b_ref, y2_ref, s_ref, q_ref):
    i = pl.program_id(0)
    yn = jnp.maximum(y_ref[...] * sc_ref[...] + sh_ref[...], 0.0)
    y2 = jnp.dot(yn, w_ref[...], preferred_element_type=jnp.float32) + b_ref[...]
    y2_ref[...] = y2

    @pl.when(i == 0)
    def _():
        s_ref[...] = jnp.zeros_like(s_ref)
        q_ref[...] = jnp.zeros_like(q_ref)

    s_ref[...] += jnp.sum(y2, axis=0, keepdims=True)
    q_ref[...] += jnp.sum(y2 * y2, axis=0, keepdims=True)


def _mm2(y1, sc1, sh1, W2, b2):
    return pl.pallas_call(
        _k_mm2,
        grid=(NBLK,),
        in_specs=[
            pl.BlockSpec((BLK, 2 * H), lambda i: (i, 0)),
            pl.BlockSpec((1, 2 * H), lambda i: (0, 0)),
            pl.BlockSpec((1, 2 * H), lambda i: (0, 0)),
            pl.BlockSpec((2 * H, H), lambda i: (0, 0)),
            pl.BlockSpec((1, H), lambda i: (0, 0)),
        ],
        out_specs=[
            pl.BlockSpec((BLK, H), lambda i: (i, 0)),
            pl.BlockSpec((1, H), lambda i: (0, 0)),
            pl.BlockSpec((1, H), lambda i: (0, 0)),
        ],
        out_shape=[
            jax.ShapeDtypeStruct((N, H), jnp.float32),
            jax.ShapeDtypeStruct((1, H), jnp.float32),
            jax.ShapeDtypeStruct((1, H), jnp.float32),
        ],
    )(y1, sc1, sh1, W2, b2)


def _k_norm(y2_ref, sc_ref, sh_ref, o_ref):
    o_ref[0] = jnp.maximum(y2_ref[...] * sc_ref[...] + sh_ref[...], 0.0)


def _norm(y2, sc2, sh2):
    cw = H // 2
    return pl.pallas_call(
        _k_norm,
        grid=(NBLK, 2),
        in_specs=[
            pl.BlockSpec((BLK, cw), lambda i, k: (i, k)),
            pl.BlockSpec((1, cw), lambda i, k: (0, k)),
            pl.BlockSpec((1, cw), lambda i, k: (0, k)),
        ],
        out_specs=pl.BlockSpec((1, BLK, cw), lambda i, k: (k, i, 0)),
        out_shape=jax.ShapeDtypeStruct((2, N, cw), jnp.float32),
    )(y2, sc2, sh2)


def _k_head(h0_ref, h1_ref, bt_ref, w1_ref, b1_ref, g_ref, bb_ref,
            w2_ref, b2_ref, o_ref, acc_ref):
    i = pl.program_id(0)

    @pl.when(i == 0)
    def _():
        acc_ref[...] = jnp.zeros_like(acc_ref)

    h = jnp.concatenate([h0_ref[0], h1_ref[0]], axis=1)
    bvals = bt_ref[0, 0, :]
    oh = (bvals[:, None] ==
          lax.broadcasted_iota(jnp.int32, (1, G), 1)).astype(jnp.float32)
    acc_ref[...] += lax.dot_general(
        oh, h, (((0,), (0,)), ((), ())), preferred_element_type=jnp.float32)

    @pl.when(i == pl.num_programs(0) - 1)
    def _():
        p = acc_ref[...]
        y = jnp.dot(p, w1_ref[...],
                    preferred_element_type=jnp.float32) + b1_ref[...]
        m = jnp.mean(y, axis=0, keepdims=True)
        v = jnp.mean(y * y, axis=0, keepdims=True) - m * m
        yn = jnp.maximum(
            g_ref[...] * (y - m) * lax.rsqrt(v + 1e-5) + bb_ref[...], 0.0)
        z = jnp.dot(yn, w2_ref[...],
                    preferred_element_type=jnp.float32) + b2_ref[...]
        ze = z - jnp.max(z, axis=1, keepdims=True)
        o_ref[...] = ze - jnp.log(jnp.sum(jnp.exp(ze), axis=1, keepdims=True))


def _head(h2, batch3, w1, b1, g, bb, w2, b2):
    cw = H // 2
    return pl.pallas_call(
        _k_head,
        grid=(NBLK,),
        in_specs=[
            pl.BlockSpec((1, BLK, cw), lambda i: (0, i, 0)),
            pl.BlockSpec((1, BLK, cw), lambda i: (1, i, 0)),
            pl.BlockSpec((1, 1, BLK), lambda i: (i, 0, 0)),
            pl.BlockSpec((H, H), lambda i: (0, 0)),
            pl.BlockSpec((1, H), lambda i: (0, 0)),
            pl.BlockSpec((1, H), lambda i: (0, 0)),
            pl.BlockSpec((1, H), lambda i: (0, 0)),
            pl.BlockSpec((H, OUT), lambda i: (0, 0)),
            pl.BlockSpec((1, OUT), lambda i: (0, 0)),
        ],
        out_specs=pl.BlockSpec((G, OUT), lambda i: (0, 0)),
        out_shape=jax.ShapeDtypeStruct((G, OUT), jnp.float32),
        scratch_shapes=[pltpu.VMEM((G, H), jnp.float32)],
    )(h2, h2, batch3, w1, b1, g, bb, w2, b2)


def kernel(x, edge_index, batch, params):
    src = edge_index[0]
    dst = edge_index[1]
    npad_e = EPAD - E
    pad_i = jnp.arange(npad_e, dtype=jnp.int32)
    srcp = jnp.concatenate([src, pad_i % N])
    dstp = jnp.concatenate([dst, N + pad_i % (NPAD - N)])
    srcp2d = srcp.reshape(-1, CH)
    src2p2d = jnp.concatenate([srcp, srcp + N]).reshape(-1, CH)
    dstp2d = dstp.reshape(-1, CH)
    batch3 = batch.reshape(NBLK, 1, BLK)

    cw = IN // 2
    h2 = jnp.stack([x[:, :cw], x[:, cw:]])
    for i in range(LAYERS):
        p = params[f"conv{i}"]
        if i == 0:
            agg = _make_seg_sum(True)(x, srcp2d, dstp2d)
        else:
            table = h2.reshape(2 * N, cw)
            agg = _make_seg_sum(False)(table, src2p2d, dstp2d)
        epsp = (1.0 + p["eps"]).reshape(1, 1)
        y1, s1, q1 = _mm1(h2, agg, p["W1"], p["b1"].reshape(1, -1),
                          epsp, 2 * cw, agg_sum=(i == 0))
        mu = s1 / N
        isg = lax.rsqrt(q1 / N - mu * mu + 1e-5)
        sc1 = p["g1"].reshape(1, -1) * isg
        sh1 = p["bt1"].reshape(1, -1) - mu * sc1
        y2, s2, q2 = _mm2(y1, sc1, sh1, p["W2"], p["b2"].reshape(1, -1))
        mu2 = s2 / N
        isg2 = lax.rsqrt(q2 / N - mu2 * mu2 + 1e-5)
        sc2 = params[f"bn{i}_g"].reshape(1, -1) * isg2
        sh2 = params[f"bn{i}_b"].reshape(1, -1) - mu2 * sc2
        h2 = _norm(y2, sc2, sh2)
        cw = H // 2

    return _head(h2, batch3, params["lin1_W"],
                 params["lin1_b"].reshape(1, -1),
                 params["bn1_g"].reshape(1, -1),
                 params["bn1_b"].reshape(1, -1),
                 params["lin2_W"], params["lin2_b"].reshape(1, -1))


# speedup vs baseline: 1.2787x; 1.0405x over previous
import functools

import jax
import jax.numpy as jnp
from jax import lax
from jax.experimental import pallas as pl
from jax.experimental.pallas import tpu as pltpu
from jax.experimental.pallas import tpu_sc as plsc

N = 10000
E = 320000
IN = 128
H = 256
OUT = 64
LAYERS = 3
G = 64

NSUB = 16
NPAD = 10240
ROWS_PER_SUB = NPAD // NSUB
CH = 128
CW = 128
EPAD = 327680

BLK = 1000
NBLK = N // BLK


@functools.cache
def _make_seg_sum(split_edges):
    mesh = plsc.VectorSubcoreMesh(core_axis_name="c", subcore_axis_name="s",
                                  num_cores=2, num_subcores=NSUB)
    ep = EPAD // 32 if split_edges else EPAD // NSUB
    nc = ep // CH
    GC = 16
    ng = nc // GC

    @functools.partial(
        pl.kernel,
        out_type=jax.ShapeDtypeStruct((2, NPAD, CW), jnp.float32),
        mesh=mesh,
        scratch_types=[
            pltpu.VMEM_SHARED((NPAD, CW), jnp.float32),
            pltpu.VMEM((2, GC, CH), jnp.int32),
            pltpu.VMEM((2, GC, CH), jnp.int32),
            pltpu.VMEM((CH, CW), jnp.float32),
            pltpu.VMEM((CH, CW), jnp.float32),
            pltpu.SemaphoreType.DMA,
            pltpu.SemaphoreType.DMA,
            pltpu.SemaphoreType.DMA,
            pltpu.SemaphoreType.DMA,
            pltpu.SemaphoreType.DMA,
        ],
    )
    def seg_sum(table_hbm, src_hbm, dst_hbm, out_hbm,
                acc, sidx, didx, rows0, rows1, sg0, sg1, ss0, ss1, si):
        cid = lax.axis_index("c")
        sid = lax.axis_index("s")
        if split_edges:
            srow = (cid * NSUB + sid) * nc
            drow = srow
        else:
            drow = sid * nc
            srow = cid * (EPAD // CH) + drow
        zoff = sid * ROWS_PER_SUB

        def wait_rows(sem):
            pltpu.make_async_copy(table_hbm.at[pl.ds(0, CH)], rows0, sem).wait()

        def wait_idx():
            pltpu.make_async_copy(src_hbm.at[pl.ds(srow, GC)],
                                  sidx.at[0], si).wait()

        pltpu.sync_copy(src_hbm.at[pl.ds(srow, GC)], sidx.at[0])
        pltpu.sync_copy(dst_hbm.at[pl.ds(drow, GC)], didx.at[0])
        pltpu.async_copy(table_hbm.at[sidx.at[0].at[0]], rows0, sg0)

        def zbody(i, _):
            r = i // (CW // 16)
            c16 = (i % (CW // 16)) * 16
            rows1[r, pl.ds(c16, 16)] = jnp.zeros((16,), jnp.float32)
            return 0
        lax.fori_loop(0, CH * (CW // 16), zbody, 0)
        for j in range(ROWS_PER_SUB // CH):
            pltpu.async_copy(rows1, acc.at[pl.ds(zoff + j * CH, CH)], si)
        for j in range(ROWS_PER_SUB // CH):
            pltpu.make_async_copy(table_hbm.at[pl.ds(0, CH)], rows1, si).wait()
        plsc.subcore_barrier()
        pltpu.async_copy(table_hbm.at[sidx.at[0].at[1]], rows1, sg1)

        for g in range(ng):
            gslot = g % 2
            sU = sidx.at[gslot]
            dU = didx.at[gslot]
            if g + 1 < ng:
                nslot = (g + 1) % 2
                pltpu.async_copy(src_hbm.at[pl.ds(srow + (g + 1) * GC, GC)],
                                 sidx.at[nslot], si)
                pltpu.async_copy(dst_hbm.at[pl.ds(drow + (g + 1) * GC, GC)],
                                 didx.at[nslot], si)

            has_next = g + 1 < ng
            nxt = sidx.at[(g + 1) % 2] if has_next else None

            def body(p, _, sU=sU, dU=dU, nxt=nxt, has_next=has_next):
                k0 = 2 * p
                k1 = k0 + 1
                last = GC // 2 - 1

                wait_rows(sg0)
                pltpu.async_copy(rows0, acc.at[dU.at[k0]], ss0, add=True)
                wait_rows(ss0)

                @pl.when(k0 + 2 < GC)
                def _():
                    pltpu.async_copy(table_hbm.at[sU.at[k0 + 2]], rows0, sg0)

                if has_next:
                    @pl.when(p == last)
                    def _():
                        wait_idx()
                        wait_idx()
                        pltpu.async_copy(table_hbm.at[nxt.at[0]], rows0, sg0)

                wait_rows(sg1)
                pltpu.async_copy(rows1, acc.at[dU.at[k1]], ss1, add=True)
                wait_rows(ss1)

                @pl.when(k1 + 2 < GC)
                def _():
                    pltpu.async_copy(table_hbm.at[sU.at[k1 + 2]], rows1, sg1)

                if has_next:
                    @pl.when(p == last)
                    def _():
                        pltpu.async_copy(table_hbm.at[nxt.at[1]], rows1, sg1)
                return 0

            lax.fori_loop(0, GC // 2, body, 0)

        plsc.subcore_barrier()
        pltpu.sync_copy(acc.at[pl.ds(zoff, ROWS_PER_SUB)],
                        out_hbm.at[cid, pl.ds(zoff, ROWS_PER_SUB)])

    return seg_sum


def _k_mm1(h0_ref, h1_ref, a0_ref, a1_ref, w_ref, b_ref, eps_ref,
           y_ref, s_ref, q_ref, *, agg_sum):
    i = pl.program_id(0)
    h = jnp.concatenate([h0_ref[0], h1_ref[0]], axis=1)
    if agg_sum:
        a = a0_ref[0] + a1_ref[0]
    else:
        a = jnp.concatenate([a0_ref[0], a1_ref[0]], axis=1)
    t = h * eps_ref[0, 0] + a
    y = jnp.dot(t, w_ref[...], preferred_element_type=jnp.float32) + b_ref[...]
    y_ref[...] = y

    @pl.when(i == 0)
    def _():
        s_ref[...] = jnp.zeros_like(s_ref)
        q_ref[...] = jnp.zeros_like(q_ref)

    s_ref[...] += jnp.sum(y, axis=0, keepdims=True)
    q_ref[...] += jnp.sum(y * y, axis=0, keepdims=True)


def _mm1(h2, agg, W1, b1, epsp, c, agg_sum):
    cw = c // 2
    return pl.pallas_call(
        functools.partial(_k_mm1, agg_sum=agg_sum),
        grid=(NBLK,),
        in_specs=[
            pl.BlockSpec((1, BLK, cw), lambda i: (0, i, 0)),
            pl.BlockSpec((1, BLK, cw), lambda i: (1, i, 0)),
            pl.BlockSpec((1, BLK, CW), lambda i: (0, i, 0)),
            pl.BlockSpec((1, BLK, CW), lambda i: (1, i, 0)),
            pl.BlockSpec((c, 2 * H), lambda i: (0, 0)),
            pl.BlockSpec((1, 2 * H), lambda i: (0, 0)),
            pl.BlockSpec(memory_space=pltpu.SMEM),
        ],
        out_specs=[
            pl.BlockSpec((BLK, 2 * H), lambda i: (i, 0)),
            pl.BlockSpec((1, 2 * H), lambda i: (0, 0)),
            pl.BlockSpec((1, 2 * H), lambda i: (0, 0)),
        ],
        out_shape=[
            jax.ShapeDtypeStruct((N, 2 * H), jnp.float32),
            jax.ShapeDtypeStruct((1, 2 * H), jnp.float32),
            jax.ShapeDtypeStruct((1, 2 * H), jnp.float32),
        ],
    )(h2, h2, agg, agg, W1, b1, epsp)


def _k_mm2(y_ref, sc_ref, sh_ref, w_ref, b_ref, y2_ref, s_ref, q_ref):
    i = pl.program_id(0)
    yn = jnp.maximum(y_ref[...] * sc_ref[...] + sh_ref[...], 0.0)
    y2 = jnp.dot(yn, w_ref[...], preferred_element_type=jnp.float32) + b_ref[...]
    y2_ref[...] = y2

    @pl.when(i == 0)
    def _():
        s_ref[...] = jnp.zeros_like(s_ref)
        q_ref[...] = jnp.zeros_like(q_ref)

    s_ref[...] += jnp.sum(y2, axis=0, keepdims=True)
    q_ref[...] += jnp.sum(y2 * y2, axis=0, keepdims=True)


def _mm2(y1, sc1, sh1, W2, b2):
    return pl.pallas_call(
        _k_mm2,
        grid=(NBLK,),
        in_specs=[
            pl.BlockSpec((BLK, 2 * H), lambda i: (i, 0)),
            pl.BlockSpec((1, 2 * H), lambda i: (0, 0)),
            pl.BlockSpec((1, 2 * H), lambda i: (0, 0)),
            pl.BlockSpec((2 * H, H), lambda i: (0, 0)),
            pl.BlockSpec((1, H), lambda i: (0, 0)),
        ],
        out_specs=[
            pl.BlockSpec((BLK, H), lambda i: (i, 0)),
            pl.BlockSpec((1, H), lambda i: (0, 0)),
            pl.BlockSpec((1, H), lambda i: (0, 0)),
        ],
        out_shape=[
            jax.ShapeDtypeStruct((N, H), jnp.float32),
            jax.ShapeDtypeStruct((1, H), jnp.float32),
            jax.ShapeDtypeStruct((1, H), jnp.float32),
        ],
    )(y1, sc1, sh1, W2, b2)


def _k_norm(y2_ref, sc_ref, sh_ref, o_ref):
    o_ref[0] = jnp.maximum(y2_ref[...] * sc_ref[...] + sh_ref[...], 0.0)


def _norm(y2, sc2, sh2):
    cw = H // 2
    return pl.pallas_call(
        _k_norm,
        grid=(NBLK, 2),
        in_specs=[
            pl.BlockSpec((BLK, cw), lambda i, k: (i, k)),
            pl.BlockSpec((1, cw), lambda i, k: (0, k)),
            pl.BlockSpec((1, cw), lambda i, k: (0, k)),
        ],
        out_specs=pl.BlockSpec((1, BLK, cw), lambda i, k: (k, i, 0)),
        out_shape=jax.ShapeDtypeStruct((2, N, cw), jnp.float32),
    )(y2, sc2, sh2)


def _k_head(h0_ref, h1_ref, bt_ref, w1_ref, b1_ref, g_ref, bb_ref,
            w2_ref, b2_ref, o_ref, acc_ref):
    i = pl.program_id(0)

    @pl.when(i == 0)
    def _():
        acc_ref[...] = jnp.zeros_like(acc_ref)

    h = jnp.concatenate([h0_ref[0], h1_ref[0]], axis=1)
    bvals = bt_ref[0, 0, :]
    oh = (bvals[:, None] ==
          lax.broadcasted_iota(jnp.int32, (1, G), 1)).astype(jnp.float32)
    acc_ref[...] += lax.dot_general(
        oh, h, (((0,), (0,)), ((), ())), preferred_element_type=jnp.float32)

    @pl.when(i == pl.num_programs(0) - 1)
    def _():
        p = acc_ref[...]
        y = jnp.dot(p, w1_ref[...],
                    preferred_element_type=jnp.float32) + b1_ref[...]
        m = jnp.mean(y, axis=0, keepdims=True)
        v = jnp.mean(y * y, axis=0, keepdims=True) - m * m
        yn = jnp.maximum(
            g_ref[...] * (y - m) * lax.rsqrt(v + 1e-5) + bb_ref[...], 0.0)
        z = jnp.dot(yn, w2_ref[...],
                    preferred_element_type=jnp.float32) + b2_ref[...]
        ze = z - jnp.max(z, axis=1, keepdims=True)
        o_ref[...] = ze - jnp.log(jnp.sum(jnp.exp(ze), axis=1, keepdims=True))


def _head(h2, batch3, w1, b1, g, bb, w2, b2):
    cw = H // 2
    return pl.pallas_call(
        _k_head,
        grid=(NBLK,),
        in_specs=[
            pl.BlockSpec((1, BLK, cw), lambda i: (0, i, 0)),
            pl.BlockSpec((1, BLK, cw), lambda i: (1, i, 0)),
            pl.BlockSpec((1, 1, BLK), lambda i: (i, 0, 0)),
            pl.BlockSpec((H, H), lambda i: (0, 0)),
            pl.BlockSpec((1, H), lambda i: (0, 0)),
            pl.BlockSpec((1, H), lambda i: (0, 0)),
            pl.BlockSpec((1, H), lambda i: (0, 0)),
            pl.BlockSpec((H, OUT), lambda i: (0, 0)),
            pl.BlockSpec((1, OUT), lambda i: (0, 0)),
        ],
        out_specs=pl.BlockSpec((G, OUT), lambda i: (0, 0)),
        out_shape=jax.ShapeDtypeStruct((G, OUT), jnp.float32),
        scratch_shapes=[pltpu.VMEM((G, H), jnp.float32)],
    )(h2, h2, batch3, w1, b1, g, bb, w2, b2)


def kernel(x, edge_index, batch, params):
    src = edge_index[0]
    dst = edge_index[1]
    npad_e = EPAD - E
    pad_i = jnp.arange(npad_e, dtype=jnp.int32)
    srcp = jnp.concatenate([src, pad_i % N])
    dstp = jnp.concatenate([dst, N + pad_i % (NPAD - N)])
    srcp2d = srcp.reshape(-1, CH)
    src2p2d = jnp.concatenate([srcp, srcp + N]).reshape(-1, CH)
    dstp2d = dstp.reshape(-1, CH)
    batch3 = batch.reshape(NBLK, 1, BLK)

    cw = IN // 2
    h2 = jnp.stack([x[:, :cw], x[:, cw:]])
    for i in range(LAYERS):
        p = params[f"conv{i}"]
        if i == 0:
            agg = _make_seg_sum(True)(x, srcp2d, dstp2d)
        else:
            table = h2.reshape(2 * N, cw)
            agg = _make_seg_sum(False)(table, src2p2d, dstp2d)
        epsp = (1.0 + p["eps"]).reshape(1, 1)
        y1, s1, q1 = _mm1(h2, agg, p["W1"], p["b1"].reshape(1, -1),
                          epsp, 2 * cw, agg_sum=(i == 0))
        mu = s1 / N
        isg = lax.rsqrt(q1 / N - mu * mu + 1e-5)
        sc1 = p["g1"].reshape(1, -1) * isg
        sh1 = p["bt1"].reshape(1, -1) - mu * sc1
        y2, s2, q2 = _mm2(y1, sc1, sh1, p["W2"], p["b2"].reshape(1, -1))
        mu2 = s2 / N
        isg2 = lax.rsqrt(q2 / N - mu2 * mu2 + 1e-5)
        sc2 = params[f"bn{i}_g"].reshape(1, -1) * isg2
        sh2 = params[f"bn{i}_b"].reshape(1, -1) - mu2 * sc2
        h2 = _norm(y2, sc2, sh2)
        cw = H // 2

    return _head(h2, batch3, params["lin1_W"],
                 params["lin1_b"].reshape(1, -1),
                 params["bn1_g"].reshape(1, -1),
                 params["bn1_b"].reshape(1, -1),
                 params["lin2_W"], params["lin2_b"].reshape(1, -1))
